# baseline (speedup 1.0000x reference)
import jax
import jax.numpy as jnp
from jax.experimental import pallas as pl
from jax.experimental.pallas import tpu as pltpu

_VOCAB = 100000
_D = 128
_BATCH = 16384
_BLK = 256
_NB = _BATCH // _BLK
_BROWS = (_VOCAB + 127) // 128


def _k1(i1s, w1_hbm, b1t, e1_out, pb_out, tbl, bb, sem):
    @pl.when(pl.program_id(0) == 0)
    def _():
        pltpu.make_async_copy(w1_hbm, tbl, sem).start()
        pltpu.make_async_copy(w1_hbm, tbl, sem).wait()

    lane = jax.lax.broadcasted_iota(jnp.int32, (1, _D), 1)
    for mi in range(_BLK):
        v = i1s[0, 0, mi]
        e1_out[mi : mi + 1, :] = tbl[v]
        bb[mi : mi + 1, :] = jnp.where(lane == (v & 127), b1t[v >> 7], 0.0)
    pb_out[:] = jnp.sum(bb[:], axis=1, keepdims=True)


def _k2(i2s, e1blk, pbblk, w2_hbm, b2t, out, tbl, e2, bb, sem):
    @pl.when(pl.program_id(0) == 0)
    def _():
        pltpu.make_async_copy(w2_hbm, tbl, sem).start()
        pltpu.make_async_copy(w2_hbm, tbl, sem).wait()

    lane = jax.lax.broadcasted_iota(jnp.int32, (1, _D), 1)
    for mi in range(_BLK):
        v = i2s[0, 0, mi]
        e2[mi : mi + 1, :] = tbl[v]
        bb[mi : mi + 1, :] = jnp.where(lane == (v & 127), b2t[v >> 7], 0.0)
    out[:] = (
        jnp.sum(e1blk[:] * e2[:] + bb[:], axis=1, keepdims=True) + pbblk[:]
    )


def kernel(i1, i2, W1, W2, b1, b2):
    w1r = W1.reshape(_VOCAB, 1, _D)
    w2r = W2.reshape(_VOCAB, 1, _D)
    pad = _BROWS * 128 - _VOCAB
    b1t = jnp.pad(b1[:, 0], (0, pad)).reshape(_BROWS, 1, 128)
    b2t = jnp.pad(b2[:, 0], (0, pad)).reshape(_BROWS, 1, 128)
    i1m = i1.reshape(_NB, 1, _BLK)
    i2m = i2.reshape(_NB, 1, _BLK)

    cp = pltpu.CompilerParams(
        dimension_semantics=("arbitrary",),
        vmem_limit_bytes=64 * 1024 * 1024,
    )
    smem_spec = pl.BlockSpec(
        (1, 1, _BLK), lambda g: (g, 0, 0), memory_space=pltpu.SMEM
    )
    btab_spec = pl.BlockSpec((_BROWS, 1, 128), lambda g: (0, 0, 0))

    e1, pb = pl.pallas_call(
        _k1,
        grid=(_NB,),
        in_specs=[
            smem_spec,
            pl.BlockSpec(memory_space=pl.ANY),
            btab_spec,
        ],
        out_specs=[
            pl.BlockSpec((_BLK, _D), lambda g: (g, 0)),
            pl.BlockSpec((_BLK, 1), lambda g: (g, 0)),
        ],
        out_shape=[
            jax.ShapeDtypeStruct((_BATCH, _D), jnp.float32),
            jax.ShapeDtypeStruct((_BATCH, 1), jnp.float32),
        ],
        scratch_shapes=[
            pltpu.VMEM((_VOCAB, 1, _D), jnp.float32),
            pltpu.VMEM((_BLK, _D), jnp.float32),
            pltpu.SemaphoreType.DMA,
        ],
        compiler_params=cp,
    )(i1m, w1r, b1t)

    out = pl.pallas_call(
        _k2,
        grid=(_NB,),
        in_specs=[
            smem_spec,
            pl.BlockSpec((_BLK, _D), lambda g: (g, 0)),
            pl.BlockSpec((_BLK, 1), lambda g: (g, 0)),
            pl.BlockSpec(memory_space=pl.ANY),
            btab_spec,
        ],
        out_specs=pl.BlockSpec((_BLK, 1), lambda g: (g, 0)),
        out_shape=jax.ShapeDtypeStruct((_BATCH, 1), jnp.float32),
        scratch_shapes=[
            pltpu.VMEM((_VOCAB, 1, _D), jnp.float32),
            pltpu.VMEM((_BLK, _D), jnp.float32),
            pltpu.VMEM((_BLK, _D), jnp.float32),
            pltpu.SemaphoreType.DMA,
        ],
        compiler_params=cp,
    )(i2m, e1, pb, w2r, b2t)
    return out
